# baseline (speedup 1.0000x reference)
import functools

import jax
import jax.numpy as jnp
from jax import lax
from jax.experimental import pallas as pl
from jax.experimental.pallas import tpu as pltpu
from jax.experimental.pallas import tpu_sc as plsc

OUT_H, OUT_W = 177, 152
N_PIX = OUT_H * OUT_W
NW = 32
NP = 864
PAD_N = NW * NP
BLK = 96
NBLK = NP // BLK
L = 16


def _sc_body(img_hbm, gx_hbm, gy_hbm, mgp_hbm, mnm_hbm, out_hbm,
             gxv, gyv, ia, ib, ic, idd, wa, wb, wc, wd,
             pa, pb, pc, pd, ob,
             gsem0, gsem1, osem0, osem1,
             *, B, H, W, C):
  flat_dim = H * W
  wid = lax.axis_index("s") * 2 + lax.axis_index("c")
  p0 = wid * NP

  pltpu.sync_copy(gx_hbm.at[pl.ds(p0, NP)], gxv)
  pltpu.sync_copy(gy_hbm.at[pl.ds(p0, NP)], gyv)
  pltpu.sync_copy(mgp_hbm.at[pl.ds(p0, NP)], wa.at[pl.ds(0, NP)])
  pltpu.sync_copy(mnm_hbm.at[pl.ds(p0, NP)], wb.at[pl.ds(0, NP)])

  def idx_body(i, _):
    sl = pl.ds(i * L, L)
    x = 0.5 * (gxv[sl] + 1.0) * jnp.float32(W)
    y = 0.5 * (gyv[sl] + 1.0) * jnp.float32(H)
    x0 = x.astype(jnp.int32)
    y0 = y.astype(jnp.int32)
    x0 = jnp.clip(x0, 0, W - 1)
    y0 = jnp.clip(y0, 0, H - 1)
    x1 = jnp.minimum(x0 + 1, W - 1)
    y1 = jnp.minimum(y0 + 1, H - 1)
    m = wa[sl] * wb[sl]
    x0f = x0.astype(jnp.float32)
    x1f = x1.astype(jnp.float32)
    y0f = y0.astype(jnp.float32)
    y1f = y1.astype(jnp.float32)
    dx0 = (x1f - x) * m
    dx1 = (x - x0f) * m
    dy0 = y1f - y
    dy1 = y - y0f
    row0 = y0 * W
    row1 = y1 * W
    ia[sl] = row0 + x0
    ib[sl] = row1 + x0
    ic[sl] = row0 + x1
    idd[sl] = row1 + x1
    wc[sl] = dx1 * dy0
    wd[sl] = dx1 * dy1
    wa_new = dx0 * dy0
    wb_new = dx0 * dy1
    wa[sl] = wa_new
    wb[sl] = wb_new
    return _

  lax.fori_loop(0, NP // L, idx_body, None)

  gsems = (gsem0, gsem1)
  osems = (osem0, osem1)

  def fire(blk, slot, sem):
    off = blk * BLK
    cps = []
    for idx_ref, buf in ((ia, pa), (ib, pb), (ic, pc), (idd, pd)):
      cp = pltpu.make_async_copy(
          img_hbm.at[idx_ref.at[pl.ds(off, BLK)]], buf.at[slot], sem)
      cp.start()
      cps.append(cp)
    return cps

  def batch_body(b, _):
    pending_g = {}
    pending_o = {0: None, 1: None}
    pending_g[0] = fire(0, 0, gsems[0])
    for blk in range(NBLK):
      slot = blk % 2
      nslot = 1 - slot
      if blk + 1 < NBLK:
        pending_g[nslot] = fire(blk + 1, nslot, gsems[nslot])
      for cp in pending_g[slot]:
        cp.wait()
      if pending_o[slot] is not None:
        pending_o[slot].wait()
      base = blk * BLK

      def comp_body(p, _):
        wav = wa[pl.ds(base + p, L)][0]
        wbv = wb[pl.ds(base + p, L)][0]
        wcv = wc[pl.ds(base + p, L)][0]
        wdv = wd[pl.ds(base + p, L)][0]
        for c in range(C // L):
          cs = pl.ds(c * L, L)
          acc = (wav * pa[slot, p, cs] + wbv * pb[slot, p, cs]
                 + wcv * pc[slot, p, cs] + wdv * pd[slot, p, cs])
          ob[slot, p, cs] = acc
        return _

      lax.fori_loop(0, BLK, comp_body, None)
      ocp = pltpu.make_async_copy(
          ob.at[slot], out_hbm.at[b, pl.ds(p0 + base, BLK)], osems[slot])
      ocp.start()
      pending_o[slot] = ocp
    for slot in (0, 1):
      if pending_o[slot] is not None:
        pending_o[slot].wait()

    def bump_body(i, _):
      sl = pl.ds(i * L, L)
      ia[sl] = ia[sl] + flat_dim
      ib[sl] = ib[sl] + flat_dim
      ic[sl] = ic[sl] + flat_dim
      idd[sl] = idd[sl] + flat_dim
      return _

    lax.fori_loop(0, NP // L, bump_body, None)
    return _

  lax.fori_loop(0, B, batch_body, None)


def kernel(image, grid, gp_mask, norm_mask):
  B, H, W, C = image.shape
  img_flat = image.reshape(B * H * W, C)
  pad = PAD_N - N_PIX
  gx = jnp.pad(grid[0], (0, pad))
  gy = jnp.pad(grid[1], (0, pad))
  mgp = jnp.pad(gp_mask.reshape(-1), (0, pad))
  mnm = jnp.pad(norm_mask.reshape(-1), (0, pad))

  mesh = plsc.VectorSubcoreMesh(core_axis_name="c", subcore_axis_name="s")
  sc_fn = functools.partial(_sc_body, B=B, H=H, W=W, C=C)
  out = pl.kernel(
      sc_fn,
      out_type=jax.ShapeDtypeStruct((B, PAD_N, C), jnp.float32),
      mesh=mesh,
      compiler_params=pltpu.CompilerParams(use_tc_tiling_on_sc=False),
      scratch_types=[
          pltpu.VMEM((NP,), jnp.float32),
          pltpu.VMEM((NP,), jnp.float32),
          pltpu.VMEM((NP,), jnp.int32),
          pltpu.VMEM((NP,), jnp.int32),
          pltpu.VMEM((NP,), jnp.int32),
          pltpu.VMEM((NP,), jnp.int32),
          pltpu.VMEM((NP + L,), jnp.float32),
          pltpu.VMEM((NP + L,), jnp.float32),
          pltpu.VMEM((NP + L,), jnp.float32),
          pltpu.VMEM((NP + L,), jnp.float32),
          pltpu.VMEM((2, BLK, C), jnp.float32),
          pltpu.VMEM((2, BLK, C), jnp.float32),
          pltpu.VMEM((2, BLK, C), jnp.float32),
          pltpu.VMEM((2, BLK, C), jnp.float32),
          pltpu.VMEM((2, BLK, C), jnp.float32),
          pltpu.SemaphoreType.DMA,
          pltpu.SemaphoreType.DMA,
          pltpu.SemaphoreType.DMA,
          pltpu.SemaphoreType.DMA,
      ],
  )(img_flat, gx, gy, mgp, mnm)
  return out[:, :N_PIX, :].reshape(B, OUT_H, OUT_W, C)
